# baseline (speedup 1.0000x reference)
import functools

import jax
import jax.numpy as jnp
from jax import lax
from jax.experimental import pallas as pl
from jax.experimental.pallas import tpu as pltpu
from jax.experimental.pallas import tpu_sc as plsc

_NC = 2
_NS = 16
_NW = _NC * _NS

_GROUP = 128
_GPC = 3
_DW = 8


def _grid_sizes(n, e):
    assert e % _GROUP == 0
    rows = e // _GROUP
    rw = rows // _NW
    rem = rows - rw * _NW
    full = rw // _GPC
    rtail = rw - full * _GPC
    rpt = -(-n // _NS)
    rpt = -(-rpt // 8) * 8
    np_ = rpt * _NS
    return rows, rw, rem, full, rtail, rpt, np_


def _worker_base(w, rw, rem):
    return w * rw + jnp.minimum(w, rem)


def _make_deg(n, e):
    rows, rw, rem, full, rtail, rpt, np_ = _grid_sizes(n, e)
    mesh = plsc.VectorSubcoreMesh(core_axis_name="c", subcore_axis_name="s")

    @functools.partial(
        pl.kernel,
        mesh=mesh,
        out_type=jax.ShapeDtypeStruct((_NC, np_, _DW), jnp.float32),
        scratch_types=[
            pltpu.VMEM((6, _GPC * _GROUP), jnp.int32),
            pltpu.VMEM((_GROUP,), jnp.int32),
            pltpu.VMEM((_GPC * _GROUP, _DW), jnp.float32),
            pltpu.VMEM_SHARED((np_, _DW), jnp.float32),
            pltpu.SemaphoreType.DMA,
            pltpu.SemaphoreType.DMA,
        ],
        compiler_params=pltpu.CompilerParams(use_tc_tiling_on_sc=False),
    )
    def deg_kernel(ei_hbm, zero_hbm, one_hbm, out_hbm, dst_v, dstt_v, one_v,
                   acc_sh, sem_i, sem_s):
        c = lax.axis_index("c")
        s = lax.axis_index("s")
        w = s * _NC + c
        base = _worker_base(w, rw, rem)
        base_e = base * _GROUP
        flat = _GPC * _GROUP

        def idx_load(k, b):
            pltpu.async_copy(ei_hbm.at[1, pl.ds(base_e + k * flat, flat)],
                             dst_v.at[b], sem_i)

        def idx_wait(b):
            pltpu.make_async_copy(ei_hbm.at[1, pl.ds(base_e, flat)],
                                  dst_v.at[b], sem_i).wait()

        def scat(b):
            pltpu.async_copy(one_v, acc_sh.at[dst_v.at[b]], sem_s, add=True)

        def scat_wait_chunk():
            pltpu.make_async_copy(one_v, acc_sh.at[dst_v.at[0]], sem_s).wait()

        pltpu.sync_copy(one_hbm, one_v)
        idx_load(0, 0)
        pltpu.sync_copy(zero_hbm.at[pl.ds(s * rpt, rpt)],
                        acc_sh.at[pl.ds(s * rpt, rpt)])
        plsc.subcore_barrier()
        idx_load(1, 1)
        idx_load(2, 2)

        def body(k, carry):
            b = k % 6
            idx_wait(b)
            scat(b)

            @pl.when(k + 3 < full)
            def _pre():
                idx_load(k + 3, (k + 3) % 6)

            @pl.when(k >= 3)
            def _ret():
                scat_wait_chunk()

            return carry

        lax.fori_loop(0, full, body, 0, unroll=False)
        for _ in range(min(3, full)):
            scat_wait_chunk()

        def single(goff):
            pltpu.sync_copy(ei_hbm.at[1, pl.ds(goff, _GROUP)], dstt_v)
            pltpu.sync_copy(one_v.at[pl.ds(0, _GROUP)], acc_sh.at[dstt_v],
                            add=True)

        for j in range(rtail):
            single(base_e + (full * _GPC + j) * _GROUP)

        @pl.when(w < rem)
        def _tail():
            single(base_e + rw * _GROUP)

        plsc.subcore_barrier()
        pltpu.sync_copy(acc_sh.at[pl.ds(s * rpt, rpt)],
                        out_hbm.at[c, pl.ds(s * rpt, rpt)])

    return deg_kernel


def _make_agg(n, e, d):
    rows, rw, rem, full, rtail, rpt, np_ = _grid_sizes(n, e)
    mesh = plsc.VectorSubcoreMesh(core_axis_name="c", subcore_axis_name="s")

    @functools.partial(
        pl.kernel,
        mesh=mesh,
        out_type=jax.ShapeDtypeStruct((_NC, np_, d), jnp.float32),
        scratch_types=[
            pltpu.VMEM((2, _GPC * _GROUP), jnp.int32),
            pltpu.VMEM((2, _GPC * _GROUP), jnp.int32),
            pltpu.VMEM((_GROUP,), jnp.int32),
            pltpu.VMEM((_GROUP,), jnp.int32),
            pltpu.VMEM((2, _GPC * _GROUP, d), jnp.float32),
            pltpu.VMEM_SHARED((np_, d), jnp.float32),
            pltpu.SemaphoreType.DMA,
            pltpu.SemaphoreType.DMA,
            pltpu.SemaphoreType.DMA,
        ],
        compiler_params=pltpu.CompilerParams(use_tc_tiling_on_sc=False),
    )
    def agg_kernel(u_hbm, ei_hbm, zero_hbm, out_hbm, src_v, dst_v, srct_v,
                   dstt_v, rows_v, acc_sh, sem_i, sem_g, sem_s):
        c = lax.axis_index("c")
        s = lax.axis_index("s")
        w = s * _NC + c
        base = _worker_base(w, rw, rem)
        base_e = base * _GROUP
        flat = _GPC * _GROUP

        def idx_load(k, b):
            off = base_e + k * flat
            pltpu.async_copy(ei_hbm.at[0, pl.ds(off, flat)], src_v.at[b], sem_i)
            pltpu.async_copy(ei_hbm.at[1, pl.ds(off, flat)], dst_v.at[b], sem_i)

        def idx_wait(b):
            pltpu.make_async_copy(ei_hbm.at[0, pl.ds(base_e, flat)],
                                  src_v.at[b], sem_i).wait()
            pltpu.make_async_copy(ei_hbm.at[1, pl.ds(base_e, flat)],
                                  dst_v.at[b], sem_i).wait()

        def gather(b):
            for j in range(_GPC):
                pltpu.async_copy(
                    u_hbm.at[src_v.at[b, pl.ds(j * _GROUP, _GROUP)]],
                    rows_v.at[b, pl.ds(j * _GROUP, _GROUP)], sem_g)

        def gather_wait(b):
            for j in range(_GPC):
                pltpu.make_async_copy(
                    u_hbm.at[src_v.at[b, pl.ds(j * _GROUP, _GROUP)]],
                    rows_v.at[b, pl.ds(j * _GROUP, _GROUP)], sem_g).wait()

        def scat(b):
            pltpu.async_copy(rows_v.at[b], acc_sh.at[dst_v.at[b]], sem_s,
                             add=True)

        def scat_wait(b):
            pltpu.make_async_copy(rows_v.at[b], acc_sh.at[dst_v.at[b]],
                                  sem_s).wait()

        pltpu.sync_copy(ei_hbm.at[0, pl.ds(base_e, flat)], src_v.at[0])
        pltpu.sync_copy(ei_hbm.at[1, pl.ds(base_e, flat)], dst_v.at[0])
        gather(0)
        idx_load(1, 1)
        pltpu.sync_copy(zero_hbm.at[pl.ds(s * rpt, rpt)],
                        acc_sh.at[pl.ds(s * rpt, rpt)])
        plsc.subcore_barrier()

        gather_wait(0)
        scat(0)
        idx_wait(1)
        gather(1)

        def body(m, carry):
            k = 2 * m + 1
            scat_wait(0)
            idx_load(k + 1, 0)
            gather_wait(1)
            scat(1)
            idx_wait(0)
            gather(0)
            scat_wait(1)
            idx_load(k + 2, 1)
            gather_wait(0)
            scat(0)
            idx_wait(1)
            gather(1)
            return carry

        pairs = (full - 2) // 2
        lax.fori_loop(0, pairs, body, 0, unroll=False)
        done = 1 + 2 * pairs
        for k in range(done, full):
            b = k % 2
            scat_wait(1 - b)
            if k + 1 < full:
                idx_load(k + 1, 1 - b)
            gather_wait(b)
            scat(b)
            if k + 1 < full:
                idx_wait(1 - b)
                gather(1 - b)
        scat_wait((full - 1) % 2)

        def single(goff):
            pltpu.sync_copy(ei_hbm.at[0, pl.ds(goff, _GROUP)], srct_v)
            pltpu.sync_copy(ei_hbm.at[1, pl.ds(goff, _GROUP)], dstt_v)
            pltpu.async_copy(u_hbm.at[srct_v],
                             rows_v.at[0, pl.ds(0, _GROUP)], sem_g).wait()
            pltpu.sync_copy(rows_v.at[0, pl.ds(0, _GROUP)],
                            acc_sh.at[dstt_v], add=True)

        for j in range(rtail):
            single(base_e + (full * _GPC + j) * _GROUP)

        @pl.when(w < rem)
        def _tail():
            single(base_e + rw * _GROUP)

        plsc.subcore_barrier()
        pltpu.sync_copy(acc_sh.at[pl.ds(s * rpt, rpt)],
                        out_hbm.at[c, pl.ds(s * rpt, rpt)])

    return agg_kernel


_R4BLK = 3128


def _init_body(deg_ref, x_ref, w_ref, dinv_ref, u_ref):
    r4 = x_ref.shape[0]
    li8 = lax.broadcasted_iota(jnp.int32, (32, 4), 0)
    jj8 = lax.broadcasted_iota(jnp.int32, (32, 4), 1)
    pick = ((li8 // 8 == jj8) & (li8 % 8 == 0)).astype(jnp.float32)
    d44 = jnp.dot(deg_ref[0] + deg_ref[1], pick,
                  preferred_element_type=jnp.float32) + 1.0
    dinv44 = lax.rsqrt(d44)
    ji = lax.broadcasted_iota(jnp.int32, (4, 128), 0)
    li = lax.broadcasted_iota(jnp.int32, (4, 128), 1)
    sel = (li // 32 == ji).astype(jnp.float32)
    xb = jnp.dot(x_ref[...], sel, preferred_element_type=jnp.float32)
    dinv = jnp.dot(dinv44, sel, preferred_element_type=jnp.float32)
    h0 = jnp.zeros((r4, 128), jnp.float32)
    for cls in range(w_ref.shape[0]):
        h0 += jnp.where(xb == float(cls), w_ref[cls:cls + 1, :], 0.0)
    dinv_ref[...] = dinv
    u_ref[...] = h0 * dinv


def _update_body(relu, rescale, p_ref, u_ref, dinv_ref, w_ref, b_ref, o_ref):
    g = (p_ref[0] + p_ref[1] + u_ref[...]) * dinv_ref[...]
    y = jnp.dot(g, w_ref[...], preferred_element_type=jnp.float32) + b_ref[...]
    if relu:
        y = jnp.maximum(y, 0.0)
    if rescale:
        y = y * dinv_ref[...]
    o_ref[...] = y


def _tc_init(deg32, x4f, w_in_t, np4):
    grid = np4 // _R4BLK
    return pl.pallas_call(
        _init_body,
        grid=(grid,),
        in_specs=[
            pl.BlockSpec((_NC, _R4BLK, 32), lambda i: (0, i, 0)),
            pl.BlockSpec((_R4BLK, 4), lambda i: (i, 0)),
            pl.BlockSpec(w_in_t.shape, lambda i: (0, 0)),
        ],
        out_specs=[
            pl.BlockSpec((_R4BLK, 128), lambda i: (i, 0)),
            pl.BlockSpec((_R4BLK, 128), lambda i: (i, 0)),
        ],
        out_shape=[
            jax.ShapeDtypeStruct((np4, 128), jnp.float32),
            jax.ShapeDtypeStruct((np4, 128), jnp.float32),
        ],
    )(deg32, x4f, w_in_t)


def _tc_update(p4, u4, dinv4, w4, b4, relu, rescale, np4):
    grid = np4 // _R4BLK
    dout = w4.shape[1]
    return pl.pallas_call(
        functools.partial(_update_body, relu, rescale),
        grid=(grid,),
        in_specs=[
            pl.BlockSpec((_NC, _R4BLK, 128), lambda i: (0, i, 0)),
            pl.BlockSpec((_R4BLK, 128), lambda i: (i, 0)),
            pl.BlockSpec((_R4BLK, 128), lambda i: (i, 0)),
            pl.BlockSpec((128, dout), lambda i: (0, 0)),
            pl.BlockSpec((1, dout), lambda i: (0, 0)),
        ],
        out_specs=pl.BlockSpec((_R4BLK, dout), lambda i: (i, 0)),
        out_shape=jax.ShapeDtypeStruct((np4, dout), jnp.float32),
    )(p4, u4, dinv4, w4, b4)


def _blockdiag4(w):
    din, dout = w.shape
    z = jnp.zeros((din, dout), w.dtype)
    return jnp.block([
        [w, z, z, z],
        [z, w, z, z],
        [z, z, w, z],
        [z, z, z, w],
    ])


def kernel(x, edge_index, num_iterations, W_in, W_shared, b_shared, W_dec, b_dec):
    n = x.shape[0]
    e = edge_index.shape[1]
    cdim, d = W_in.shape
    assert d == 32 and n % 4 == 0
    rows, rw, rem, full, rtail, rpt, np_ = _grid_sizes(n, e)
    np4 = np_ // 4

    zero_rows = jnp.zeros((np_, d), jnp.float32)
    zero_deg = jnp.zeros((np_, _DW), jnp.float32)
    one_g = jnp.ones((_GPC * _GROUP, _DW), jnp.float32)

    deg_kernel = _make_deg(n, e)
    agg_kernel = _make_agg(n, e, d)

    deg = deg_kernel(edge_index, zero_deg, one_g)
    deg32 = deg.reshape(_NC, np4, 32)

    x4f = jnp.pad(x.astype(jnp.float32).reshape(n // 4, 4),
                  ((0, np4 - n // 4), (0, 0)))
    w_in_t = jnp.tile(W_in, (1, 4))
    dinv4, u4 = _tc_init(deg32, x4f, w_in_t, np4)

    w4 = _blockdiag4(W_shared)
    b4 = jnp.tile(b_shared.reshape(1, d), (1, 4))
    wd4 = _blockdiag4(W_dec)
    bd4 = jnp.tile(b_dec.reshape(1, cdim), (1, 4))

    def one_iter(_, u4):
        p = agg_kernel(u4.reshape(np_, d), edge_index, zero_rows)
        p4 = p.reshape(_NC, np4, 128)
        return _tc_update(p4, u4, dinv4, w4, b4, True, True, np4)

    u4 = lax.fori_loop(0, num_iterations, one_iter, u4)

    p = agg_kernel(u4.reshape(np_, d), edge_index, zero_rows)
    p4 = p.reshape(_NC, np4, 128)
    y4 = _tc_update(p4, u4, dinv4, wd4, bd4, False, False, np4)
    return y4[:n // 4].reshape(n, cdim)


# speedup vs baseline: 1.0353x; 1.0353x over previous
import functools

import jax
import jax.numpy as jnp
from jax import lax
from jax.experimental import pallas as pl
from jax.experimental.pallas import tpu as pltpu
from jax.experimental.pallas import tpu_sc as plsc

_NC = 2
_NS = 16
_NW = _NC * _NS

_GROUP = 128
_GPC = 3
_DW = 8


def _grid_sizes(n, e):
    assert e % _GROUP == 0
    rows = e // _GROUP
    rw = rows // _NW
    rem = rows - rw * _NW
    full = rw // _GPC
    rtail = rw - full * _GPC
    rpt = -(-n // _NS)
    rpt = -(-rpt // 8) * 8
    np_ = rpt * _NS
    return rows, rw, rem, full, rtail, rpt, np_


def _worker_base(w, rw, rem):
    return w * rw + jnp.minimum(w, rem)


def _make_deg(n, e):
    rows, rw, rem, full, rtail, rpt, np_ = _grid_sizes(n, e)
    mesh = plsc.VectorSubcoreMesh(core_axis_name="c", subcore_axis_name="s")

    @functools.partial(
        pl.kernel,
        mesh=mesh,
        out_type=jax.ShapeDtypeStruct((_NC, np_, _DW), jnp.float32),
        scratch_types=[
            pltpu.VMEM((6, _GPC * _GROUP), jnp.int32),
            pltpu.VMEM((_GROUP,), jnp.int32),
            pltpu.VMEM((_GPC * _GROUP, _DW), jnp.float32),
            pltpu.VMEM_SHARED((np_, _DW), jnp.float32),
            pltpu.SemaphoreType.DMA,
            pltpu.SemaphoreType.DMA,
        ],
        compiler_params=pltpu.CompilerParams(use_tc_tiling_on_sc=False),
    )
    def deg_kernel(ei_hbm, zero_hbm, one_hbm, out_hbm, dst_v, dstt_v, one_v,
                   acc_sh, sem_i, sem_s):
        c = lax.axis_index("c")
        s = lax.axis_index("s")
        w = s * _NC + c
        base = _worker_base(w, rw, rem)
        base_e = base * _GROUP
        flat = _GPC * _GROUP

        def idx_load(k, b):
            pltpu.async_copy(ei_hbm.at[1, pl.ds(base_e + k * flat, flat)],
                             dst_v.at[b], sem_i)

        def idx_wait(b):
            pltpu.make_async_copy(ei_hbm.at[1, pl.ds(base_e, flat)],
                                  dst_v.at[b], sem_i).wait()

        def scat(b):
            pltpu.async_copy(one_v, acc_sh.at[dst_v.at[b]], sem_s, add=True)

        def scat_wait_chunk():
            pltpu.make_async_copy(one_v, acc_sh.at[dst_v.at[0]], sem_s).wait()

        pltpu.sync_copy(one_hbm, one_v)
        idx_load(0, 0)
        pltpu.sync_copy(zero_hbm.at[pl.ds(s * rpt, rpt)],
                        acc_sh.at[pl.ds(s * rpt, rpt)])
        plsc.subcore_barrier()
        idx_load(1, 1)
        idx_load(2, 2)

        def body(k, carry):
            b = k % 6
            idx_wait(b)
            scat(b)

            @pl.when(k + 3 < full)
            def _pre():
                idx_load(k + 3, (k + 3) % 6)

            @pl.when(k >= 3)
            def _ret():
                scat_wait_chunk()

            return carry

        lax.fori_loop(0, full, body, 0, unroll=False)
        for _ in range(min(3, full)):
            scat_wait_chunk()

        def single(goff):
            pltpu.sync_copy(ei_hbm.at[1, pl.ds(goff, _GROUP)], dstt_v)
            pltpu.sync_copy(one_v.at[pl.ds(0, _GROUP)], acc_sh.at[dstt_v],
                            add=True)

        for j in range(rtail):
            single(base_e + (full * _GPC + j) * _GROUP)

        @pl.when(w < rem)
        def _tail():
            single(base_e + rw * _GROUP)

        plsc.subcore_barrier()
        pltpu.sync_copy(acc_sh.at[pl.ds(s * rpt, rpt)],
                        out_hbm.at[c, pl.ds(s * rpt, rpt)])

    return deg_kernel


def _make_agg(n, e, d):
    rows, rw, rem, full, rtail, rpt, np_ = _grid_sizes(n, e)
    mesh = plsc.VectorSubcoreMesh(core_axis_name="c", subcore_axis_name="s")

    @functools.partial(
        pl.kernel,
        mesh=mesh,
        out_type=jax.ShapeDtypeStruct((_NC, np_, d), jnp.float32),
        scratch_types=[
            pltpu.VMEM((2, _GPC, _GROUP), jnp.int32),
            pltpu.VMEM((2, _GPC, _GROUP), jnp.int32),
            pltpu.VMEM((2, _GPC, _GROUP, d), jnp.float32),
            pltpu.VMEM_SHARED((np_, d), jnp.float32),
            pltpu.SemaphoreType.DMA,
            pltpu.SemaphoreType.DMA,
            pltpu.SemaphoreType.DMA,
        ],
        compiler_params=pltpu.CompilerParams(use_tc_tiling_on_sc=False),
    )
    def agg_kernel(u_hbm, ei_hbm, zero_hbm, out_hbm, src_v, dst_v, rows_v,
                   acc_sh, sem_i, sem_g, sem_s):
        c = lax.axis_index("c")
        s = lax.axis_index("s")
        w = s * _NC + c
        base = _worker_base(w, rw, rem)

        def idx_load(k, b):
            rb = base + k * _GPC
            pltpu.async_copy(ei_hbm.at[0, pl.ds(rb, _GPC)], src_v.at[b], sem_i)
            pltpu.async_copy(ei_hbm.at[1, pl.ds(rb, _GPC)], dst_v.at[b], sem_i)

        def idx_wait(b):
            pltpu.make_async_copy(ei_hbm.at[0, pl.ds(base, _GPC)],
                                  src_v.at[b], sem_i).wait()
            pltpu.make_async_copy(ei_hbm.at[1, pl.ds(base, _GPC)],
                                  dst_v.at[b], sem_i).wait()

        def gather(b):
            for j in range(_GPC):
                pltpu.async_copy(u_hbm.at[src_v.at[b, j]], rows_v.at[b, j],
                                 sem_g)

        def gather_wait(b):
            for j in range(_GPC):
                pltpu.make_async_copy(u_hbm.at[src_v.at[b, j]],
                                      rows_v.at[b, j], sem_g).wait()

        def scat(b):
            for j in range(_GPC):
                pltpu.async_copy(rows_v.at[b, j], acc_sh.at[dst_v.at[b, j]],
                                 sem_s, add=True)

        def scat_wait(b):
            for j in range(_GPC):
                pltpu.make_async_copy(rows_v.at[b, j],
                                      acc_sh.at[dst_v.at[b, j]], sem_s).wait()

        pltpu.sync_copy(ei_hbm.at[0, pl.ds(base, _GPC)], src_v.at[0])
        pltpu.sync_copy(ei_hbm.at[1, pl.ds(base, _GPC)], dst_v.at[0])
        gather(0)
        idx_load(1, 1)
        pltpu.sync_copy(zero_hbm.at[pl.ds(s * rpt, rpt)],
                        acc_sh.at[pl.ds(s * rpt, rpt)])
        plsc.subcore_barrier()

        gather_wait(0)
        scat(0)
        idx_wait(1)
        gather(1)

        def body(m, carry):
            k = 2 * m + 1
            scat_wait(0)
            idx_load(k + 1, 0)
            gather_wait(1)
            scat(1)
            idx_wait(0)
            gather(0)
            scat_wait(1)
            idx_load(k + 2, 1)
            gather_wait(0)
            scat(0)
            idx_wait(1)
            gather(1)
            return carry

        pairs = (full - 2) // 2
        lax.fori_loop(0, pairs, body, 0, unroll=False)
        done = 1 + 2 * pairs
        for k in range(done, full):
            b = k % 2
            scat_wait(1 - b)
            if k + 1 < full:
                idx_load(k + 1, 1 - b)
            gather_wait(b)
            scat(b)
            if k + 1 < full:
                idx_wait(1 - b)
                gather(1 - b)
        scat_wait((full - 1) % 2)

        def single(rb):
            pltpu.sync_copy(ei_hbm.at[0, pl.ds(rb, 1)], src_v.at[0, pl.ds(0, 1)])
            pltpu.sync_copy(ei_hbm.at[1, pl.ds(rb, 1)], dst_v.at[0, pl.ds(0, 1)])
            pltpu.async_copy(u_hbm.at[src_v.at[0, 0]], rows_v.at[0, 0],
                             sem_g).wait()
            pltpu.sync_copy(rows_v.at[0, 0], acc_sh.at[dst_v.at[0, 0]], add=True)

        for j in range(rtail):
            single(base + full * _GPC + j)

        @pl.when(w < rem)
        def _tail():
            single(base + rw)

        plsc.subcore_barrier()
        pltpu.sync_copy(acc_sh.at[pl.ds(s * rpt, rpt)],
                        out_hbm.at[c, pl.ds(s * rpt, rpt)])

    return agg_kernel


_R4BLK = 3128


def _init_body(deg_ref, x_ref, w_ref, dinv_ref, u_ref):
    r4 = x_ref.shape[0]
    li8 = lax.broadcasted_iota(jnp.int32, (32, 4), 0)
    jj8 = lax.broadcasted_iota(jnp.int32, (32, 4), 1)
    pick = ((li8 // 8 == jj8) & (li8 % 8 == 0)).astype(jnp.float32)
    d44 = jnp.dot(deg_ref[0] + deg_ref[1], pick,
                  preferred_element_type=jnp.float32) + 1.0
    dinv44 = lax.rsqrt(d44)
    ji = lax.broadcasted_iota(jnp.int32, (4, 128), 0)
    li = lax.broadcasted_iota(jnp.int32, (4, 128), 1)
    sel = (li // 32 == ji).astype(jnp.float32)
    xb = jnp.dot(x_ref[...], sel, preferred_element_type=jnp.float32)
    dinv = jnp.dot(dinv44, sel, preferred_element_type=jnp.float32)
    h0 = jnp.zeros((r4, 128), jnp.float32)
    for cls in range(w_ref.shape[0]):
        h0 += jnp.where(xb == float(cls), w_ref[cls:cls + 1, :], 0.0)
    dinv_ref[...] = dinv
    u_ref[...] = h0 * dinv


def _update_body(relu, rescale, p_ref, u_ref, dinv_ref, w_ref, b_ref, o_ref):
    g = (p_ref[0] + p_ref[1] + u_ref[...]) * dinv_ref[...]
    y = jnp.dot(g, w_ref[...], preferred_element_type=jnp.float32) + b_ref[...]
    if relu:
        y = jnp.maximum(y, 0.0)
    if rescale:
        y = y * dinv_ref[...]
    o_ref[...] = y


def _tc_init(deg32, x4f, w_in_t, np4):
    grid = np4 // _R4BLK
    return pl.pallas_call(
        _init_body,
        grid=(grid,),
        in_specs=[
            pl.BlockSpec((_NC, _R4BLK, 32), lambda i: (0, i, 0)),
            pl.BlockSpec((_R4BLK, 4), lambda i: (i, 0)),
            pl.BlockSpec(w_in_t.shape, lambda i: (0, 0)),
        ],
        out_specs=[
            pl.BlockSpec((_R4BLK, 128), lambda i: (i, 0)),
            pl.BlockSpec((_R4BLK, 128), lambda i: (i, 0)),
        ],
        out_shape=[
            jax.ShapeDtypeStruct((np4, 128), jnp.float32),
            jax.ShapeDtypeStruct((np4, 128), jnp.float32),
        ],
    )(deg32, x4f, w_in_t)


def _tc_update(p4, u4, dinv4, w4, b4, relu, rescale, np4):
    grid = np4 // _R4BLK
    dout = w4.shape[1]
    return pl.pallas_call(
        functools.partial(_update_body, relu, rescale),
        grid=(grid,),
        in_specs=[
            pl.BlockSpec((_NC, _R4BLK, 128), lambda i: (0, i, 0)),
            pl.BlockSpec((_R4BLK, 128), lambda i: (i, 0)),
            pl.BlockSpec((_R4BLK, 128), lambda i: (i, 0)),
            pl.BlockSpec((128, dout), lambda i: (0, 0)),
            pl.BlockSpec((1, dout), lambda i: (0, 0)),
        ],
        out_specs=pl.BlockSpec((_R4BLK, dout), lambda i: (i, 0)),
        out_shape=jax.ShapeDtypeStruct((np4, dout), jnp.float32),
    )(p4, u4, dinv4, w4, b4)


def _blockdiag4(w):
    din, dout = w.shape
    z = jnp.zeros((din, dout), w.dtype)
    return jnp.block([
        [w, z, z, z],
        [z, w, z, z],
        [z, z, w, z],
        [z, z, z, w],
    ])


def kernel(x, edge_index, num_iterations, W_in, W_shared, b_shared, W_dec, b_dec):
    n = x.shape[0]
    e = edge_index.shape[1]
    cdim, d = W_in.shape
    assert d == 32 and n % 4 == 0
    rows, rw, rem, full, rtail, rpt, np_ = _grid_sizes(n, e)
    np4 = np_ // 4

    ei3 = edge_index.reshape(2, rows, _GROUP)
    zero_rows = jnp.zeros((np_, d), jnp.float32)
    zero_deg = jnp.zeros((np_, _DW), jnp.float32)
    one_g = jnp.ones((_GPC * _GROUP, _DW), jnp.float32)

    deg_kernel = _make_deg(n, e)
    agg_kernel = _make_agg(n, e, d)

    deg = deg_kernel(edge_index, zero_deg, one_g)
    deg32 = deg.reshape(_NC, np4, 32)

    x4f = jnp.pad(x.astype(jnp.float32).reshape(n // 4, 4),
                  ((0, np4 - n // 4), (0, 0)))
    w_in_t = jnp.tile(W_in, (1, 4))
    dinv4, u4 = _tc_init(deg32, x4f, w_in_t, np4)

    w4 = _blockdiag4(W_shared)
    b4 = jnp.tile(b_shared.reshape(1, d), (1, 4))
    wd4 = _blockdiag4(W_dec)
    bd4 = jnp.tile(b_dec.reshape(1, cdim), (1, 4))

    def one_iter(_, u4):
        p = agg_kernel(u4.reshape(np_, d), ei3, zero_rows)
        p4 = p.reshape(_NC, np4, 128)
        return _tc_update(p4, u4, dinv4, w4, b4, True, True, np4)

    u4 = lax.fori_loop(0, num_iterations, one_iter, u4)

    p = agg_kernel(u4.reshape(np_, d), ei3, zero_rows)
    p4 = p.reshape(_NC, np4, 128)
    y4 = _tc_update(p4, u4, dinv4, wd4, bd4, False, False, np4)
    return y4[:n // 4].reshape(n, cdim)
